# baseline (speedup 1.0000x reference)
import jax
import jax.numpy as jnp
from jax.experimental import pallas as pl
from jax.experimental.pallas import tpu as pltpu


def _produce(f_ref, o_ref, sem):
    cp = pltpu.make_async_copy(f_ref, o_ref, sem)
    cp.start()
    cp.wait()


def kernel(history_data, future_data, batch_seen, epoch, train, dummy_param):
    return pl.pallas_call(
        _produce,
        out_shape=jax.ShapeDtypeStruct(future_data.shape, jnp.float32),
        in_specs=[pl.BlockSpec(memory_space=pl.ANY)],
        out_specs=pl.BlockSpec(memory_space=pl.ANY),
        scratch_shapes=[pltpu.SemaphoreType.DMA],
    )(future_data)


# speedup vs baseline: 1.6709x; 1.6709x over previous
import functools

import jax
import jax.numpy as jnp
from jax import lax
from jax.experimental import pallas as pl
from jax.experimental.pallas import tpu as pltpu
from jax.experimental.pallas import tpu_sc as plsc

_MESH = plsc.VectorSubcoreMesh(core_axis_name="c", subcore_axis_name="s")


def _sc_body(f_hbm, o_hbm, buf):
    wid = lax.axis_index("s") * 2 + lax.axis_index("c")

    @pl.when(wid == 0)
    def _():
        pltpu.sync_copy(f_hbm, buf)
        pltpu.sync_copy(buf, o_hbm)


def kernel(history_data, future_data, batch_seen, epoch, train, dummy_param):
    n = future_data.size
    flat = future_data.reshape(n)
    run = functools.partial(
        pl.kernel,
        out_type=jax.ShapeDtypeStruct((n,), jnp.float32),
        mesh=_MESH,
        scratch_types=[pltpu.VMEM((n,), jnp.float32)],
    )(_sc_body)
    return run(flat).reshape(future_data.shape)


# speedup vs baseline: 3.0068x; 1.7995x over previous
import jax
import jax.numpy as jnp
from jax.experimental import pallas as pl
from jax.experimental.pallas import tpu as pltpu


def _combine(d_ref, f_ref, o_ref):
    o_ref[...] = f_ref[...] + 0.0 * d_ref[0]


def kernel(history_data, future_data, batch_seen, epoch, train, dummy_param):
    b, w, f, c = future_data.shape
    return pl.pallas_call(
        _combine,
        grid=(w,),
        out_shape=jax.ShapeDtypeStruct(future_data.shape, jnp.float32),
        in_specs=[
            pl.BlockSpec(memory_space=pltpu.SMEM),
            pl.BlockSpec((b, 1, f, c), lambda t: (0, t, 0, 0)),
        ],
        out_specs=pl.BlockSpec((b, 1, f, c), lambda t: (0, t, 0, 0)),
    )(dummy_param, future_data)


# speedup vs baseline: 5.4549x; 1.8142x over previous
import jax
import jax.numpy as jnp
from jax.experimental import pallas as pl
from jax.experimental.pallas import tpu as pltpu


def _combine(d_ref, f_ref, o_ref):
    o_ref[...] = f_ref[...] + 0.0 * d_ref[0]


def kernel(history_data, future_data, batch_seen, epoch, train, dummy_param):
    n = future_data.size
    flat = future_data.reshape(n)
    out = pl.pallas_call(
        _combine,
        out_shape=jax.ShapeDtypeStruct((n,), jnp.float32),
        in_specs=[
            pl.BlockSpec(memory_space=pltpu.SMEM),
            pl.BlockSpec(memory_space=pltpu.VMEM),
        ],
    )(dummy_param, flat)
    return out.reshape(future_data.shape)


# speedup vs baseline: 7.3375x; 1.3451x over previous
import jax
import jax.numpy as jnp
from jax.experimental import pallas as pl
from jax.experimental.pallas import tpu as pltpu


def _combine(d_ref, f_ref, o_ref):
    o_ref[...] = f_ref[...] + 0.0 * d_ref[0]


def kernel(history_data, future_data, batch_seen, epoch, train, dummy_param):
    b, w, f, c = future_data.shape
    x = future_data.transpose(1, 3, 0, 2).reshape(w, c, f)
    out = pl.pallas_call(
        _combine,
        out_shape=jax.ShapeDtypeStruct((w, c, f), jnp.float32),
        in_specs=[
            pl.BlockSpec(memory_space=pltpu.SMEM),
            pl.BlockSpec(memory_space=pltpu.VMEM),
        ],
    )(dummy_param, x)
    return out.reshape(w, c, b, f).transpose(2, 0, 3, 1)


# speedup vs baseline: 18.0775x; 2.4637x over previous
import jax
import jax.numpy as jnp
from jax.experimental import pallas as pl
from jax.experimental.pallas import tpu as pltpu


def _combine(d_ref, f_ref, o_ref):
    o_ref[...] = f_ref[...] + 0.0 * d_ref[0]


def kernel(history_data, future_data, batch_seen, epoch, train, dummy_param):
    b, w, f, c = future_data.shape
    x = future_data.transpose(1, 3, 0, 2)
    out = pl.pallas_call(
        _combine,
        out_shape=jax.ShapeDtypeStruct((w, c, b, f), jnp.float32),
        in_specs=[
            pl.BlockSpec(memory_space=pltpu.SMEM),
            pl.BlockSpec(memory_space=pltpu.VMEM),
        ],
    )(dummy_param, x)
    return out.transpose(2, 0, 3, 1)


# speedup vs baseline: 25.0927x; 1.3881x over previous
import jax
import jax.numpy as jnp
from jax.experimental import pallas as pl
from jax.experimental.pallas import tpu as pltpu


def _combine(f_ref, o_ref):
    o_ref[...] = f_ref[...]


def kernel(history_data, future_data, batch_seen, epoch, train, dummy_param):
    b, w, f, c = future_data.shape
    x = future_data.transpose(1, 3, 0, 2)
    out = pl.pallas_call(
        _combine,
        out_shape=jax.ShapeDtypeStruct((w, c, b, f), jnp.float32),
        in_specs=[pl.BlockSpec(memory_space=pltpu.VMEM)],
    )(x)
    return out.transpose(2, 0, 3, 1)
